# Initial kernel scaffold; baseline (speedup 1.0000x reference)
import jax
import jax.numpy as jnp
from jax.experimental import pallas as pl


def kernel(x, table):
    raise NotImplementedError("write your pallas kernel here")


# speedup vs baseline: 1.3761x; 1.3761x over previous
import functools

import jax
import jax.numpy as jnp
from jax import lax
from jax.experimental import pallas as pl
from jax.experimental.pallas import tpu as pltpu
from jax.experimental.pallas import tpu_sc as plsc

_D = 1024
_B = 4 * 4096
_SCALE = 32.0
_NC = 2
_NS = 16
_NW = _NC * _NS
_BPW = _B // _NW
_CHUNK = 32
_NCHUNK = _BPW // _CHUNK
_LANES = 16


def _emb_body(x_hbm, table_hbm, out_hbm, idx_v, buf0, buf1, gsem0, gsem1):
    wid = lax.axis_index("s") * _NC + lax.axis_index("c")
    base = wid * _BPW
    pltpu.sync_copy(x_hbm.at[pl.ds(base, _BPW)], idx_v)

    bufs = (buf0, buf1)
    gsems = (gsem0, gsem1)

    def gstart(g, b):
        pltpu.async_copy(
            table_hbm.at[idx_v.at[pl.ds(g * _CHUNK, _CHUNK)]],
            bufs[b],
            gsems[b],
        )

    def gwait(b):
        pltpu.make_async_copy(
            table_hbm.at[idx_v.at[pl.ds(0, _CHUNK)]],
            bufs[b],
            gsems[b],
        ).wait()

    def process(g, b):
        gwait(b)
        buf = bufs[b]

        def row_body(r, carry):
            for j in range(_D // _LANES):
                sl = pl.ds(j * _LANES, _LANES)
                buf[r, sl] = buf[r, sl] * _SCALE
            return carry

        lax.fori_loop(0, _CHUNK, row_body, 0)
        pltpu.sync_copy(buf, out_hbm.at[pl.ds(base + g * _CHUNK, _CHUNK)])

    gstart(0, 0)

    def pair(i, carry):
        g = i * 2
        gstart(g + 1, 1)
        process(g, 0)

        @pl.when(i + 1 < _NCHUNK // 2)
        def _():
            gstart(g + 2, 0)

        process(g + 1, 1)
        return carry

    lax.fori_loop(0, _NCHUNK // 2, pair, 0)


@jax.jit
def _emb(x_flat, table):
    mesh = plsc.VectorSubcoreMesh(core_axis_name="c", subcore_axis_name="s")
    run = functools.partial(
        pl.kernel,
        mesh=mesh,
        out_type=jax.ShapeDtypeStruct((_B, _D), jnp.float32),
        scratch_types=[
            pltpu.VMEM((_BPW,), jnp.int32),
            pltpu.VMEM((_CHUNK, _D), jnp.float32),
            pltpu.VMEM((_CHUNK, _D), jnp.float32),
            pltpu.SemaphoreType.DMA,
            pltpu.SemaphoreType.DMA,
        ],
    )(_emb_body)
    return run(x_flat, table)


def kernel(x, table):
    x_flat = x.reshape(-1).astype(jnp.int32)
    out = _emb(x_flat, table)
    return out.reshape(x.shape + (_D,))


# speedup vs baseline: 1.4694x; 1.0678x over previous
import functools

import jax
import jax.numpy as jnp
from jax import lax
from jax.experimental import pallas as pl
from jax.experimental.pallas import tpu as pltpu
from jax.experimental.pallas import tpu_sc as plsc

_D = 1024
_B = 4 * 4096
_SCALE = 32.0
_NC = 2
_NS = 16
_NW = _NC * _NS
_BPW = _B // _NW
_CHUNK = 32
_NCHUNK = _BPW // _CHUNK
_NBUF = 3
_LANES = 16


def _emb_body(x_hbm, table_hbm, out_hbm, idx_v,
              buf0, buf1, buf2, gsem0, gsem1, gsem2, osem0, osem1, osem2):
    wid = lax.axis_index("s") * _NC + lax.axis_index("c")
    base = wid * _BPW
    pltpu.sync_copy(x_hbm.at[pl.ds(base, _BPW)], idx_v)

    bufs = (buf0, buf1, buf2)
    gsems = (gsem0, gsem1, gsem2)
    osems = (osem0, osem1, osem2)

    def gstart(g, b):
        pltpu.async_copy(
            table_hbm.at[idx_v.at[pl.ds(g * _CHUNK, _CHUNK)]],
            bufs[b],
            gsems[b],
        )

    def gwait(b):
        pltpu.make_async_copy(
            table_hbm.at[idx_v.at[pl.ds(0, _CHUNK)]],
            bufs[b],
            gsems[b],
        ).wait()

    def ostart(g, b):
        pltpu.async_copy(
            bufs[b],
            out_hbm.at[pl.ds(base + g * _CHUNK, _CHUNK)],
            osems[b],
        )

    def owait(b):
        pltpu.make_async_copy(
            bufs[b],
            out_hbm.at[pl.ds(0, _CHUNK)],
            osems[b],
        ).wait()

    def scale(b):
        buf = bufs[b]

        def row_body(r, carry):
            for j in range(_D // _LANES):
                sl = pl.ds(j * _LANES, _LANES)
                buf[r, sl] = buf[r, sl] * _SCALE
            return carry

        lax.fori_loop(0, _CHUNK, row_body, 0)

    gstart(0, 0)
    gstart(1, 1)
    for g in range(_NCHUNK):
        b = g % _NBUF
        gwait(b)
        scale(b)
        ostart(g, b)
        nxt = g + _NBUF - 1
        if nxt < _NCHUNK:
            bb = nxt % _NBUF
            if nxt >= _NBUF:
                owait(bb)
            gstart(nxt, bb)
    for g in range(_NCHUNK - _NBUF, _NCHUNK):
        owait(g % _NBUF)


@jax.jit
def _emb(x_flat, table):
    mesh = plsc.VectorSubcoreMesh(core_axis_name="c", subcore_axis_name="s")
    run = functools.partial(
        pl.kernel,
        mesh=mesh,
        out_type=jax.ShapeDtypeStruct((_B, _D), jnp.float32),
        scratch_types=[
            pltpu.VMEM((_BPW,), jnp.int32),
            pltpu.VMEM((_CHUNK, _D), jnp.float32),
            pltpu.VMEM((_CHUNK, _D), jnp.float32),
            pltpu.VMEM((_CHUNK, _D), jnp.float32),
            pltpu.SemaphoreType.DMA,
            pltpu.SemaphoreType.DMA,
            pltpu.SemaphoreType.DMA,
            pltpu.SemaphoreType.DMA,
            pltpu.SemaphoreType.DMA,
            pltpu.SemaphoreType.DMA,
        ],
    )(_emb_body)
    return run(x_flat, table)


def kernel(x, table):
    x_flat = x.reshape(-1).astype(jnp.int32)
    out = _emb(x_flat, table)
    return out.reshape(x.shape + (_D,))


# speedup vs baseline: 1.6087x; 1.0948x over previous
import functools

import jax
import jax.numpy as jnp
from jax import lax
from jax.experimental import pallas as pl
from jax.experimental.pallas import tpu as pltpu
from jax.experimental.pallas import tpu_sc as plsc

_D = 1024
_B = 4 * 4096
_SCALE = 32.0
_NC = 2
_NS = 16
_NW = _NC * _NS
_BPW = _B // _NW
_CHUNK = 32
_NCHUNK = _BPW // _CHUNK
_LANES = 16
_NTRIPLE = 5


def _emb_body(x_hbm, table_hbm, out_hbm, idx_v,
              buf0, buf1, buf2, gsem0, gsem1, gsem2, osem0, osem1, osem2):
    wid = lax.axis_index("s") * _NC + lax.axis_index("c")
    base = wid * _BPW
    pltpu.sync_copy(x_hbm.at[pl.ds(base, _BPW)], idx_v)

    bufs = (buf0, buf1, buf2)
    gsems = (gsem0, gsem1, gsem2)
    osems = (osem0, osem1, osem2)

    def gstart(c, b):
        pltpu.async_copy(
            table_hbm.at[idx_v.at[pl.ds(c * _CHUNK, _CHUNK)]],
            bufs[b],
            gsems[b],
        )

    def gwait(b):
        pltpu.make_async_copy(
            table_hbm.at[idx_v.at[pl.ds(0, _CHUNK)]],
            bufs[b],
            gsems[b],
        ).wait()

    def ostart(c, b):
        pltpu.async_copy(
            bufs[b],
            out_hbm.at[pl.ds(base + c * _CHUNK, _CHUNK)],
            osems[b],
        )

    def owait(b):
        pltpu.make_async_copy(
            bufs[b],
            out_hbm.at[pl.ds(0, _CHUNK)],
            osems[b],
        ).wait()

    def scale(b):
        buf = bufs[b]

        def row_body(r, carry):
            @plsc.parallel_loop(0, _D // _LANES, step=1, unroll=8)
            def _(j):
                sl = pl.ds(j * _LANES, _LANES)
                buf[r, sl] = buf[r, sl] * _SCALE

            return carry

        lax.fori_loop(0, _CHUNK, row_body, 0)

    gstart(0, 0)
    gstart(1, 1)

    def triple(i, carry):
        c = i * 3
        gwait(0)
        scale(0)
        ostart(c, 0)

        @pl.when(i >= 1)
        def _():
            owait(2)

        gstart(c + 2, 2)
        gwait(1)
        scale(1)
        ostart(c + 1, 1)
        owait(0)
        gstart(c + 3, 0)
        gwait(2)
        scale(2)
        ostart(c + 2, 2)

        @pl.when(i <= _NTRIPLE - 2)
        def _():
            owait(1)
            gstart(c + 4, 1)

        return carry

    lax.fori_loop(0, _NTRIPLE, triple, 0)

    gwait(0)
    scale(0)
    ostart(_NCHUNK - 1, 0)
    owait(1)
    owait(2)
    owait(0)


@jax.jit
def _emb(x_flat, table):
    mesh = plsc.VectorSubcoreMesh(core_axis_name="c", subcore_axis_name="s")
    run = functools.partial(
        pl.kernel,
        mesh=mesh,
        out_type=jax.ShapeDtypeStruct((_B, _D), jnp.float32),
        scratch_types=[
            pltpu.VMEM((_BPW,), jnp.int32),
            pltpu.VMEM((_CHUNK, _D), jnp.float32),
            pltpu.VMEM((_CHUNK, _D), jnp.float32),
            pltpu.VMEM((_CHUNK, _D), jnp.float32),
            pltpu.SemaphoreType.DMA,
            pltpu.SemaphoreType.DMA,
            pltpu.SemaphoreType.DMA,
            pltpu.SemaphoreType.DMA,
            pltpu.SemaphoreType.DMA,
            pltpu.SemaphoreType.DMA,
        ],
    )(_emb_body)
    return run(x_flat, table)


def kernel(x, table):
    x_flat = x.reshape(-1).astype(jnp.int32)
    out = _emb(x_flat, table)
    return out.reshape(x.shape + (_D,))
